# Initial kernel scaffold; baseline (speedup 1.0000x reference)
import functools

import jax
import jax.numpy as jnp
from jax import lax
from jax.experimental import pallas as pl
from jax.experimental.pallas import tpu as pltpu
from jax.experimental.pallas import tpu_sc as plsc

_B, _N_AT, _N_NBH = 8, 10000, 64
_NC, _NS, _L = 2, 16, 16
_NW = _NC * _NS
_WPB = _NW // _B
_APW = _N_AT // _WPB
_CHUNK = 250
_NCHUNK = _APW // _CHUNK


def _inv_sqrt(s):
    i = lax.bitcast_convert_type(s, jnp.int32)
    i = jnp.int32(0x5F3759DF) - lax.shift_right_arithmetic(i, 1)
    y = lax.bitcast_convert_type(i, jnp.float32)
    hs = 0.5 * s
    y = y * (1.5 - hs * y * y)
    y = y * (1.5 - hs * y * y)
    return y


def kernel(positions, neighbors):
    mesh = plsc.VectorSubcoreMesh(core_axis_name="c", subcore_axis_name="s")

    @functools.partial(
        pl.kernel,
        out_type=jax.ShapeDtypeStruct((_B, _N_AT, _N_NBH), jnp.float32),
        mesh=mesh,
        scratch_types=[
            pltpu.VMEM((_N_AT, 3), jnp.float32),
            pltpu.VMEM((_CHUNK, _N_NBH), jnp.int32),
            pltpu.VMEM((_CHUNK, _N_NBH), jnp.float32),
        ],
    )
    def _k(pos_hbm, nbr_hbm, out_hbm, pos_v, nbr_v, out_v):
        wid = lax.axis_index("s") * _NC + lax.axis_index("c")
        b = wid // _WPB
        base = (wid % _WPB) * _APW
        pltpu.sync_copy(pos_hbm.at[b], pos_v)

        c0 = jnp.zeros((_L,), jnp.int32)
        c1 = jnp.full((_L,), 1, jnp.int32)
        c2 = jnp.full((_L,), 2, jnp.int32)

        def chunk_body(g, carry):
            a0 = base + g * _CHUNK
            pltpu.sync_copy(nbr_hbm.at[b, pl.ds(a0, _CHUNK)], nbr_v)

            def atom_body(a, inner):
                i = a0 + a
                sx = pos_v[i, 0]
                sy = pos_v[i, 1]
                sz = pos_v[i, 2]
                for j in range(_N_NBH // _L):
                    idx = nbr_v[a, pl.ds(j * _L, _L)]
                    nx = plsc.load_gather(pos_v, [idx, c0])
                    ny = plsc.load_gather(pos_v, [idx, c1])
                    nz = plsc.load_gather(pos_v, [idx, c2])
                    dx = nx - sx
                    dy = ny - sy
                    dz = nz - sz
                    s = dx * dx + dy * dy + dz * dz
                    s = jnp.maximum(s, 1e-30)
                    r = _inv_sqrt(s)
                    d = s * r
                    out_v[a, pl.ds(j * _L, _L)] = 1.0 / (d + 1e-8)
                return inner

            lax.fori_loop(0, _CHUNK, atom_body, 0)
            pltpu.sync_copy(out_v, out_hbm.at[b, pl.ds(a0, _CHUNK)])
            return carry

        lax.fori_loop(0, _NCHUNK, chunk_body, 0)

    return _k(positions, neighbors)


# speedup vs baseline: 171.5004x; 171.5004x over previous
import functools

import jax
import jax.numpy as jnp
from jax import lax
from jax.experimental import pallas as pl
from jax.experimental.pallas import tpu as pltpu
from jax.experimental.pallas import tpu_sc as plsc

_B, _N_AT, _N_NBH = 8, 10000, 64
_NC, _NS, _L = 2, 16, 16
_NW = _NC * _NS
_WPB = _NW // _B
_APW = _N_AT // _WPB
_CHUNK = 250
_NCHUNK = _APW // _CHUNK
_CW = _CHUNK * _N_NBH
_WW = _APW * _N_NBH


def _inv_sqrt(s):
    i = lax.bitcast_convert_type(s, jnp.int32)
    i = jnp.int32(0x5F3759DF) - lax.shift_right_arithmetic(i, 1)
    y = lax.bitcast_convert_type(i, jnp.float32)
    hs = 0.5 * s
    y = y * (1.5 - hs * y * y)
    y = y * (1.5 - hs * y * y)
    return y


def kernel(positions, neighbors):
    mesh = plsc.VectorSubcoreMesh(core_axis_name="c", subcore_axis_name="s")

    @functools.partial(
        pl.kernel,
        out_type=jax.ShapeDtypeStruct((_B * _N_AT * _N_NBH,), jnp.float32),
        mesh=mesh,
        compiler_params=pltpu.CompilerParams(needs_layout_passes=False),
        scratch_types=[
            pltpu.VMEM((_N_AT,), jnp.float32),
            pltpu.VMEM((_N_AT,), jnp.float32),
            pltpu.VMEM((_N_AT,), jnp.float32),
            pltpu.VMEM((_CW,), jnp.int32),
            pltpu.VMEM((_CW,), jnp.float32),
        ],
    )
    def _k(pos_hbm, nbr_hbm, out_hbm, px_v, py_v, pz_v, nbr_v, out_v):
        wid = lax.axis_index("s") * _NC + lax.axis_index("c")
        b = wid // _WPB
        abase = (wid % _WPB) * _APW
        wbase = wid * _WW
        pbase = b * (3 * _N_AT)
        pltpu.sync_copy(pos_hbm.at[pl.ds(pbase, _N_AT)], px_v)
        pltpu.sync_copy(pos_hbm.at[pl.ds(pbase + _N_AT, _N_AT)], py_v)
        pltpu.sync_copy(pos_hbm.at[pl.ds(pbase + 2 * _N_AT, _N_AT)], pz_v)

        def chunk_body(g, carry):
            cbase = wbase + g * _CW
            pltpu.sync_copy(nbr_hbm.at[pl.ds(cbase, _CW)], nbr_v)

            def atom_body(a, inner):
                i = abase + g * _CHUNK + a
                si = jnp.full((_L,), 0, jnp.int32) + i
                sx = plsc.load_gather(px_v, [si])
                sy = plsc.load_gather(py_v, [si])
                sz = plsc.load_gather(pz_v, [si])
                for j in range(_N_NBH // _L):
                    o = a * _N_NBH + j * _L
                    idx = nbr_v[pl.ds(o, _L)]
                    nx = plsc.load_gather(px_v, [idx])
                    ny = plsc.load_gather(py_v, [idx])
                    nz = plsc.load_gather(pz_v, [idx])
                    dx = nx - sx
                    dy = ny - sy
                    dz = nz - sz
                    s = dx * dx + dy * dy + dz * dz
                    s = jnp.maximum(s, 1e-30)
                    r = _inv_sqrt(s)
                    d = s * r
                    out_v[pl.ds(o, _L)] = 1.0 / (d + 1e-8)
                return inner

            lax.fori_loop(0, _CHUNK, atom_body, 0)
            pltpu.sync_copy(out_v, out_hbm.at[pl.ds(cbase, _CW)])
            return carry

        lax.fori_loop(0, _NCHUNK, chunk_body, 0)

    pos_soa = positions.transpose(0, 2, 1).reshape(-1)
    out = _k(pos_soa, neighbors.reshape(-1))
    return out.reshape(_B, _N_AT, _N_NBH)


# speedup vs baseline: 381.0101x; 2.2216x over previous
import functools

import jax
import jax.numpy as jnp
from jax import lax
from jax.experimental import pallas as pl
from jax.experimental.pallas import tpu as pltpu
from jax.experimental.pallas import tpu_sc as plsc

_B, _N_AT, _N_NBH = 8, 10000, 64
_NC, _NS, _L = 2, 16, 16
_NW = _NC * _NS
_WPB = _NW // _B
_APW = _N_AT // _WPB
_CHUNK = 250
_NCHUNK = _APW // _CHUNK
_CW = _CHUNK * _N_NBH
_WW = _APW * _N_NBH


def _inv_sqrt(s):
    i = lax.bitcast_convert_type(s, jnp.int32)
    i = jnp.int32(0x5F3759DF) - lax.shift_right_arithmetic(i, 1)
    y = lax.bitcast_convert_type(i, jnp.float32)
    hs = 0.5 * s
    y = y * (1.5 - hs * y * y)
    y = y * (1.5 - hs * y * y)
    return y


def kernel(positions, neighbors):
    mesh = plsc.VectorSubcoreMesh(core_axis_name="c", subcore_axis_name="s")

    @functools.partial(
        pl.kernel,
        out_type=jax.ShapeDtypeStruct((_B * _N_AT * _N_NBH,), jnp.float32),
        mesh=mesh,
        compiler_params=pltpu.CompilerParams(needs_layout_passes=False),
        scratch_types=[
            pltpu.VMEM((_N_AT,), jnp.float32),
            pltpu.VMEM((_N_AT,), jnp.float32),
            pltpu.VMEM((_N_AT,), jnp.float32),
            pltpu.VMEM((_CW,), jnp.int32),
            pltpu.VMEM((_CW,), jnp.float32),
        ],
    )
    def _k(pos_hbm, nbr_hbm, out_hbm, px_v, py_v, pz_v, nbr_v, out_v):
        wid = lax.axis_index("s") * _NC + lax.axis_index("c")
        b = wid // _WPB
        abase = (wid % _WPB) * _APW
        wbase = wid * _WW
        pbase = b * (3 * _N_AT)
        pltpu.sync_copy(pos_hbm.at[pl.ds(pbase, _N_AT)], px_v)
        pltpu.sync_copy(pos_hbm.at[pl.ds(pbase + _N_AT, _N_AT)], py_v)
        pltpu.sync_copy(pos_hbm.at[pl.ds(pbase + 2 * _N_AT, _N_AT)], pz_v)

        def chunk_body(g, carry):
            cbase = wbase + g * _CW
            pltpu.sync_copy(nbr_hbm.at[pl.ds(cbase, _CW)], nbr_v)

            @plsc.parallel_loop(0, _CHUNK, unroll=2)
            def atom_body(a):
                i = abase + g * _CHUNK + a
                si = jnp.full((_L,), 0, jnp.int32) + i
                sx = plsc.load_gather(px_v, [si])
                sy = plsc.load_gather(py_v, [si])
                sz = plsc.load_gather(pz_v, [si])
                for j in range(_N_NBH // _L):
                    o = a * _N_NBH + j * _L
                    idx = nbr_v[pl.ds(o, _L)]
                    nx = plsc.load_gather(px_v, [idx])
                    ny = plsc.load_gather(py_v, [idx])
                    nz = plsc.load_gather(pz_v, [idx])
                    dx = nx - sx
                    dy = ny - sy
                    dz = nz - sz
                    s = dx * dx + dy * dy + dz * dz
                    s = jnp.maximum(s, 1e-30)
                    r = _inv_sqrt(s)
                    d = s * r
                    out_v[pl.ds(o, _L)] = 1.0 / (d + 1e-8)
            pltpu.sync_copy(out_v, out_hbm.at[pl.ds(cbase, _CW)])
            return carry

        lax.fori_loop(0, _NCHUNK, chunk_body, 0)

    pos_soa = positions.transpose(0, 2, 1).reshape(-1)
    out = _k(pos_soa, neighbors.reshape(-1))
    return out.reshape(_B, _N_AT, _N_NBH)


# speedup vs baseline: 391.3029x; 1.0270x over previous
import functools

import jax
import jax.numpy as jnp
from jax import lax
from jax.experimental import pallas as pl
from jax.experimental.pallas import tpu as pltpu
from jax.experimental.pallas import tpu_sc as plsc

_B, _N_AT, _N_NBH = 8, 10000, 64
_NC, _NS, _L = 2, 16, 16
_NW = _NC * _NS
_WPB = _NW // _B
_APW = _N_AT // _WPB
_CHUNK = 250
_NCHUNK = _APW // _CHUNK
_CW = _CHUNK * _N_NBH
_WW = _APW * _N_NBH


def _inv_sqrt(s):
    i = lax.bitcast_convert_type(s, jnp.int32)
    i = jnp.int32(0x5F3759DF) - lax.shift_right_arithmetic(i, 1)
    y = lax.bitcast_convert_type(i, jnp.float32)
    y = y * (1.5 - 0.5 * s * y * y)
    return y


def kernel(positions, neighbors):
    mesh = plsc.VectorSubcoreMesh(core_axis_name="c", subcore_axis_name="s")

    @functools.partial(
        pl.kernel,
        out_type=jax.ShapeDtypeStruct((_B * _N_AT * _N_NBH,), jnp.float32),
        mesh=mesh,
        compiler_params=pltpu.CompilerParams(needs_layout_passes=False),
        scratch_types=[
            pltpu.VMEM((_N_AT,), jnp.float32),
            pltpu.VMEM((_N_AT,), jnp.float32),
            pltpu.VMEM((_N_AT,), jnp.float32),
            pltpu.VMEM((_CW,), jnp.int32),
            pltpu.VMEM((_CW,), jnp.float32),
        ],
    )
    def _k(pos_hbm, nbr_hbm, out_hbm, px_v, py_v, pz_v, nbr_v, out_v):
        wid = lax.axis_index("s") * _NC + lax.axis_index("c")
        b = wid // _WPB
        abase = (wid % _WPB) * _APW
        wbase = wid * _WW
        pbase = b * (3 * _N_AT)
        pltpu.sync_copy(pos_hbm.at[pl.ds(pbase, _N_AT)], px_v)
        pltpu.sync_copy(pos_hbm.at[pl.ds(pbase + _N_AT, _N_AT)], py_v)
        pltpu.sync_copy(pos_hbm.at[pl.ds(pbase + 2 * _N_AT, _N_AT)], pz_v)

        def chunk_body(g, carry):
            cbase = wbase + g * _CW
            pltpu.sync_copy(nbr_hbm.at[pl.ds(cbase, _CW)], nbr_v)

            @plsc.parallel_loop(0, _CHUNK, unroll=2)
            def atom_body(a):
                i = abase + g * _CHUNK + a
                si = jnp.full((_L,), 0, jnp.int32) + i
                sx = plsc.load_gather(px_v, [si])
                sy = plsc.load_gather(py_v, [si])
                sz = plsc.load_gather(pz_v, [si])
                for j in range(_N_NBH // _L):
                    o = a * _N_NBH + j * _L
                    idx = nbr_v[pl.ds(o, _L)]
                    nx = plsc.load_gather(px_v, [idx])
                    ny = plsc.load_gather(py_v, [idx])
                    nz = plsc.load_gather(pz_v, [idx])
                    dx = nx - sx
                    dy = ny - sy
                    dz = nz - sz
                    s = dx * dx + dy * dy + dz * dz
                    s = jnp.maximum(s, 1e-30)
                    r = _inv_sqrt(s)
                    out_v[pl.ds(o, _L)] = r * (1.0 - 1e-8 * r)
            pltpu.sync_copy(out_v, out_hbm.at[pl.ds(cbase, _CW)])
            return carry

        lax.fori_loop(0, _NCHUNK, chunk_body, 0)

    pos_soa = positions.transpose(0, 2, 1).reshape(-1)
    out = _k(pos_soa, neighbors.reshape(-1))
    return out.reshape(_B, _N_AT, _N_NBH)
